# baseline (speedup 1.0000x reference)
import functools

import jax
import jax.numpy as jnp
from jax.experimental import pallas as pl
from jax.experimental.pallas import tpu as pltpu

Q = 1024
D = 128
K = 100000
KT = 2048
G = (K + KT - 1) // KT
KP = G * KT


def _semidual_kernel(x_ref, y_ref, psi_ref, out1_ref, out2_ref, acc_ref, psum_ref):
    k = pl.program_id(0)

    @pl.when(k == 0)
    def _init():
        acc_ref[...] = jnp.full((Q, 1), 3.0e38, dtype=jnp.float32)
        psum_ref[0, 0] = 0.0

    x = x_ref[...]
    y = y_ref[...]
    psi = psi_ref[0]

    xy = jax.lax.dot_general(
        x, y, (((1,), (1,)), ((), ())), preferred_element_type=jnp.float32
    )
    ones = jnp.ones((1, D), dtype=jnp.float32)
    y2 = jax.lax.dot_general(
        ones, y * y, (((1,), (1,)), ((), ())), preferred_element_type=jnp.float32
    )
    col = jax.lax.broadcasted_iota(jnp.int32, (1, KT), 1) + k * KT
    b = jnp.where(col < K, y2 - psi, 3.0e38)

    s = b - 2.0 * xy
    acc_ref[...] = jnp.minimum(acc_ref[...], jnp.min(s, axis=1, keepdims=True))
    psum_ref[0, 0] += jnp.sum(jnp.where(col < K, psi, 0.0))

    @pl.when(k == G - 1)
    def _fini():
        x2 = jnp.sum(x * x, axis=1, keepdims=True)
        out1_ref[0, 0] = jnp.sum(acc_ref[...] + x2) * (1.0 / Q)
        out2_ref[0, 0] = psum_ref[0, 0] * (1.0 / K)


@jax.jit
def _semidual(inputx, inputy, psi):
    y_pad = jnp.pad(inputy, ((0, KP - K), (0, 0)))
    psi_pad = jnp.pad(psi, (0, KP - K)).reshape(G, 1, KT)
    out1, out2 = pl.pallas_call(
        _semidual_kernel,
        grid=(G,),
        in_specs=[
            pl.BlockSpec((Q, D), lambda k: (0, 0)),
            pl.BlockSpec((KT, D), lambda k: (k, 0)),
            pl.BlockSpec((1, 1, KT), lambda k: (k, 0, 0)),
        ],
        out_specs=[
            pl.BlockSpec(memory_space=pltpu.SMEM),
            pl.BlockSpec(memory_space=pltpu.SMEM),
        ],
        out_shape=[
            jax.ShapeDtypeStruct((1, 1), jnp.float32),
            jax.ShapeDtypeStruct((1, 1), jnp.float32),
        ],
        scratch_shapes=[
            pltpu.VMEM((Q, 1), jnp.float32),
            pltpu.SMEM((1, 1), jnp.float32),
        ],
        compiler_params=pltpu.CompilerParams(
            dimension_semantics=("arbitrary",),
        ),
    )(inputx, y_pad, psi_pad)
    return out1[0, 0], out2[0, 0]


def kernel(inputx, inputy, psi):
    return _semidual(inputx, inputy, psi)
